# baseline (speedup 1.0000x reference)
import functools

import jax
import jax.numpy as jnp
from jax.experimental import pallas as pl
from jax.experimental.pallas import tpu as pltpu

_TOPK = 30
_BITS = 30


def _fuse_mask_kernel(beta_ref, a_ref, b_ref, c_ref, o_ref, x_ref, g_ref,
                      *, k, blk):
    i = pl.program_id(0)
    x = (beta_ref[0] * a_ref[...] + beta_ref[1] * b_ref[...]
         + beta_ref[2] * c_ref[...])
    rows = jax.lax.broadcasted_iota(jnp.int32, x.shape, 0) + i * blk
    cols = jax.lax.broadcasted_iota(jnp.int32, x.shape, 1)
    x = jnp.where(rows == cols, 0.0, x)
    x_ref[...] = x

    nrows, n = x.shape
    n8 = n // 8
    g = x_ref[:, 0:n8]
    for j in range(1, 8):
        g = jnp.maximum(g, x_ref[:, j * n8:(j + 1) * n8])
    g_ref[...] = g

    def coarse_body(it, lo):
        bit = _BITS - 1 - it
        cand = lo | (1 << bit)
        t = jax.lax.bitcast_convert_type(cand, jnp.float32)
        cnt = jnp.sum((g_ref[...] >= t).astype(jnp.int32), axis=1,
                      keepdims=True)
        return jnp.where(cnt >= k, cand, lo)

    lo = jax.lax.fori_loop(
        0, _BITS, coarse_body, jnp.zeros((nrows, 1), jnp.int32))
    rowmax = jnp.max(g_ref[...], axis=1, keepdims=True)
    hi = jax.lax.bitcast_convert_type(rowmax, jnp.int32) + 1

    def fine_cond(carry):
        lo, hi = carry
        return jnp.max(hi - lo) > 1

    def fine_body(carry):
        lo, hi = carry
        mid = (lo + hi) // 2
        t = jax.lax.bitcast_convert_type(mid, jnp.float32)
        y = x_ref[...]
        cnt = jnp.sum((y >= t).astype(jnp.int32), axis=1, keepdims=True)
        ok = cnt >= k
        return jnp.where(ok, mid, lo), jnp.where(ok, hi, mid)

    lo, hi = jax.lax.while_loop(fine_cond, fine_body, (lo, hi))
    t = jax.lax.bitcast_convert_type(lo, jnp.float32)
    y = x_ref[...]
    o_ref[...] = jnp.where(y >= t, y, 0.0).astype(jnp.bfloat16)


def _sym_kernel(a_ref, b_ref, o_ref):
    o_ref[...] = jnp.maximum(a_ref[...], b_ref[...].T).astype(jnp.float32)


def _sparsify(a, b, c, beta, k):
    n = a.shape[0]
    blk = 256
    s = pl.pallas_call(
        functools.partial(_fuse_mask_kernel, k=k, blk=blk),
        grid=(n // blk,),
        in_specs=[
            pl.BlockSpec(memory_space=pltpu.SMEM),
            pl.BlockSpec((blk, n), lambda i: (i, 0)),
            pl.BlockSpec((blk, n), lambda i: (i, 0)),
            pl.BlockSpec((blk, n), lambda i: (i, 0)),
        ],
        out_specs=pl.BlockSpec((blk, n), lambda i: (i, 0)),
        out_shape=jax.ShapeDtypeStruct((n, n), jnp.bfloat16),
        scratch_shapes=[pltpu.VMEM((blk, n), jnp.float32),
                        pltpu.VMEM((blk, n // 8), jnp.float32)],
    )(beta, a, b, c)
    t = 512
    return pl.pallas_call(
        _sym_kernel,
        grid=(n // t, n // t),
        in_specs=[
            pl.BlockSpec((t, t), lambda i, j: (i, j)),
            pl.BlockSpec((t, t), lambda i, j: (j, i)),
        ],
        out_specs=pl.BlockSpec((t, t), lambda i, j: (i, j)),
        out_shape=jax.ShapeDtypeStruct((n, n), jnp.float32),
    )(s, s)


def kernel(mm_f, mm_s, mm_g, dd_t, dd_s, dd_g, att_m, att_d):
    beta_m = jax.nn.softmax(att_m.reshape(3))
    beta_d = jax.nn.softmax(att_d.reshape(3))
    m_out = _sparsify(mm_f, mm_s, mm_g, beta_m, _TOPK)
    d_out = _sparsify(dd_t, dd_s, dd_g, beta_d, _TOPK)
    return (m_out, d_out)


# speedup vs baseline: 1.2138x; 1.2138x over previous
import functools

import jax
import jax.numpy as jnp
from jax.experimental import pallas as pl
from jax.experimental.pallas import tpu as pltpu

_TOPK = 30
_BITS = 30


def _fuse_mask_kernel(beta_ref, a_ref, b_ref, c_ref, o_ref, x_ref,
                      *, k, blk):
    i = pl.program_id(0)
    x = (beta_ref[0] * a_ref[...] + beta_ref[1] * b_ref[...]
         + beta_ref[2] * c_ref[...])
    rows = jax.lax.broadcasted_iota(jnp.int32, x.shape, 0) + i * blk
    cols = jax.lax.broadcasted_iota(jnp.int32, x.shape, 1)
    x = jnp.where(rows == cols, 0.0, x)
    x_ref[...] = x

    nrows = x.shape[0]

    def body(it, lo):
        bit = _BITS - 1 - it
        cand = lo | (1 << bit)
        t = jax.lax.bitcast_convert_type(cand, jnp.float32)
        y = x_ref[...]
        cnt = jnp.sum((y >= t).astype(jnp.int32), axis=1, keepdims=True)
        return jnp.where(cnt >= k, cand, lo)

    lo = jax.lax.fori_loop(
        0, _BITS, body, jnp.zeros((nrows, 1), jnp.int32))
    t = jax.lax.bitcast_convert_type(lo, jnp.float32)
    y = x_ref[...]
    o_ref[...] = jnp.where(y >= t, y, 0.0).astype(jnp.bfloat16)


def _sym_kernel(a_ref, b_ref, o_ref):
    o_ref[...] = jnp.maximum(a_ref[...], b_ref[...].T).astype(jnp.float32)


def _sparsify(a, b, c, beta, k):
    n = a.shape[0]
    blk = 256
    s = pl.pallas_call(
        functools.partial(_fuse_mask_kernel, k=k, blk=blk),
        grid=(n // blk,),
        in_specs=[
            pl.BlockSpec(memory_space=pltpu.SMEM),
            pl.BlockSpec((blk, n), lambda i: (i, 0)),
            pl.BlockSpec((blk, n), lambda i: (i, 0)),
            pl.BlockSpec((blk, n), lambda i: (i, 0)),
        ],
        out_specs=pl.BlockSpec((blk, n), lambda i: (i, 0)),
        out_shape=jax.ShapeDtypeStruct((n, n), jnp.bfloat16),
        scratch_shapes=[pltpu.VMEM((blk, n), jnp.float32)],
    )(beta, a, b, c)
    t = 512
    return pl.pallas_call(
        _sym_kernel,
        grid=(n // t, n // t),
        in_specs=[
            pl.BlockSpec((t, t), lambda i, j: (i, j)),
            pl.BlockSpec((t, t), lambda i, j: (j, i)),
        ],
        out_specs=pl.BlockSpec((t, t), lambda i, j: (i, j)),
        out_shape=jax.ShapeDtypeStruct((n, n), jnp.float32),
    )(s, s)


def kernel(mm_f, mm_s, mm_g, dd_t, dd_s, dd_g, att_m, att_d):
    beta_m = jax.nn.softmax(att_m.reshape(3))
    beta_d = jax.nn.softmax(att_d.reshape(3))
    m_out = _sparsify(mm_f, mm_s, mm_g, beta_m, _TOPK)
    d_out = _sparsify(dd_t, dd_s, dd_g, beta_d, _TOPK)
    return (m_out, d_out)
